# baseline (device time: 362070 ns/iter reference)
import jax
import jax.numpy as jnp
from jax import lax
from jax.experimental import pallas as pl
from jax.experimental.pallas import tpu as pltpu

B, S, H, Dh, Dr = 4, 256, 32, 128, 64
D = 4096
M = B * S
NH = H // 2
QH = NH * Dh
QRH = NH * Dr

_VMEM_LIMIT = 100 * 1024 * 1024
_MESH = pl.DeviceIdType.MESH
_F32 = jnp.float32
_BF16 = jnp.bfloat16


def _sel(i):
    return jnp.reshape(i, (1,)).astype(jnp.int32)


def _mm_body(a_ref, b_ref, o_ref):
    o_ref[...] = jnp.dot(a_ref[...], b_ref[...], preferred_element_type=_F32)


def _matmul(a, b, block_n):
    m, k = a.shape
    k2, n = b.shape
    assert k == k2 and n % block_n == 0
    return pl.pallas_call(
        _mm_body,
        grid=(n // block_n,),
        in_specs=[
            pl.BlockSpec((m, k), lambda j: (0, 0)),
            pl.BlockSpec((k, block_n), lambda j: (0, j)),
        ],
        out_specs=pl.BlockSpec((m, block_n), lambda j: (0, j)),
        out_shape=jax.ShapeDtypeStruct((m, n), _F32),
        compiler_params=pltpu.CompilerParams(vmem_limit_bytes=_VMEM_LIMIT),
    )(a, b)


def _mm_half_body(s_ref, a_ref, b_ref, o_ref):
    o_ref[...] = jnp.dot(a_ref[...], b_ref[...], preferred_element_type=_F32)


def _matmul_half(a, b, sel, n_half, block_n):
    m, k = a.shape
    nblk = n_half // block_n
    grid_spec = pltpu.PrefetchScalarGridSpec(
        num_scalar_prefetch=1,
        grid=(nblk,),
        in_specs=[
            pl.BlockSpec((m, k), lambda j, s: (0, 0)),
            pl.BlockSpec((k, block_n), lambda j, s: (0, s[0] * nblk + j)),
        ],
        out_specs=pl.BlockSpec((m, block_n), lambda j, s: (0, j)),
    )
    return pl.pallas_call(
        _mm_half_body,
        grid_spec=grid_spec,
        out_shape=jax.ShapeDtypeStruct((m, n_half), _F32),
        compiler_params=pltpu.CompilerParams(vmem_limit_bytes=_VMEM_LIMIT),
    )(_sel(sel), a, b)


_KV_STEPS = 8


_LANES = 4


def _proj_body(s_ref, c_ref, wuk_ref, wuv_ref, x_ref, wq_ref,
               k_ref, v_ref, q_ref,
               sk, sv, rk, rv, sem_sk, sem_rk, sem_sv, sem_rv):
    j = pl.program_id(0)
    my_x = lax.axis_index("x")
    my_y = lax.axis_index("y")
    partner = (1 - my_x, my_y)

    def rdma_k(i):
        return pltpu.make_async_remote_copy(
            src_ref=sk.at[i], dst_ref=rk.at[i],
            send_sem=sem_sk.at[i], recv_sem=sem_rk.at[i],
            device_id=partner, device_id_type=_MESH)

    def rdma_v(i):
        return pltpu.make_async_remote_copy(
            src_ref=sv.at[i], dst_ref=rv.at[i],
            send_sem=sem_sv.at[i], recv_sem=sem_rv.at[i],
            device_id=partner, device_id_type=_MESH)

    w = QH // _LANES

    @pl.when(j == 0)
    def _():
        kp = jnp.dot(c_ref[...], wuk_ref[...], preferred_element_type=_F32)
        vp = jnp.dot(c_ref[...], wuv_ref[...], preferred_element_type=_F32)
        k_ref[...] = kp
        v_ref[...] = vp
        for i in range(_LANES):
            sk[i] = kp[:, i * w:(i + 1) * w].astype(_BF16)
            sv[i] = vp[:, i * w:(i + 1) * w].astype(_BF16)
        barrier = pltpu.get_barrier_semaphore()
        pl.semaphore_signal(barrier, inc=1, device_id=partner,
                            device_id_type=_MESH)
        pl.semaphore_wait(barrier, 1)
        for i in range(_LANES):
            rdma_k(i).start()
            rdma_v(i).start()

    q_ref[...] = jnp.dot(x_ref[...], wq_ref[...], preferred_element_type=_F32)

    @pl.when(j == _KV_STEPS - 1)
    def _():
        for i in range(_LANES):
            rdma_k(i).wait()
            rdma_v(i).wait()
        for i in range(_LANES):
            k_ref[:, i * w:(i + 1) * w] = (
                k_ref[:, i * w:(i + 1) * w] + rk[i].astype(_F32))
            v_ref[:, i * w:(i + 1) * w] = (
                v_ref[:, i * w:(i + 1) * w] + rv[i].astype(_F32))


def _proj_kv_allreduce(my_y, c, wuk, wuv, x2, wq):
    bq = QH // _KV_STEPS
    dc = c.shape[1]
    grid_spec = pltpu.PrefetchScalarGridSpec(
        num_scalar_prefetch=1,
        grid=(_KV_STEPS,),
        in_specs=[
            pl.BlockSpec((M, dc), lambda j, s: (0, 0)),
            pl.BlockSpec((dc, QH), lambda j, s: (0, s[0])),
            pl.BlockSpec((dc, QH), lambda j, s: (0, s[0])),
            pl.BlockSpec((M, D), lambda j, s: (0, 0)),
            pl.BlockSpec((D, bq), lambda j, s: (0, s[0] * _KV_STEPS + j)),
        ],
        out_specs=[
            pl.BlockSpec((M, QH), lambda j, s: (0, 0)),
            pl.BlockSpec((M, QH), lambda j, s: (0, 0)),
            pl.BlockSpec((M, bq), lambda j, s: (0, j)),
        ],
        scratch_shapes=[
            pltpu.VMEM((_LANES, M, QH // _LANES), _BF16),
            pltpu.VMEM((_LANES, M, QH // _LANES), _BF16),
            pltpu.VMEM((_LANES, M, QH // _LANES), _BF16),
            pltpu.VMEM((_LANES, M, QH // _LANES), _BF16),
            pltpu.SemaphoreType.DMA((_LANES,)),
            pltpu.SemaphoreType.DMA((_LANES,)),
            pltpu.SemaphoreType.DMA((_LANES,)),
            pltpu.SemaphoreType.DMA((_LANES,)),
        ],
    )
    return pl.pallas_call(
        _proj_body,
        grid_spec=grid_spec,
        out_shape=[
            jax.ShapeDtypeStruct((M, QH), _F32),
            jax.ShapeDtypeStruct((M, QH), _F32),
            jax.ShapeDtypeStruct((M, QH), _F32),
        ],
        compiler_params=pltpu.CompilerParams(collective_id=0,
                                             vmem_limit_bytes=_VMEM_LIMIT),
    )(_sel(my_y), c, wuk, wuv, x2, wq)


_SCALE = (Dh + Dr) ** -0.5


def _attn_body(q_ref, qr_ref, k_ref, kr_ref, v_ref, o_ref):
    dn = (((1,), (1,)), ((), ()))
    s = lax.dot_general(q_ref[...], k_ref[...], dn,
                        preferred_element_type=_F32)
    s = s + lax.dot_general(qr_ref[...], kr_ref[...], dn,
                            preferred_element_type=_F32)
    s = s * _SCALE
    m = jnp.max(s, axis=-1, keepdims=True)
    p = jnp.exp(s - m)
    p = p / jnp.sum(p, axis=-1, keepdims=True)
    o_ref[...] = jnp.dot(p, v_ref[...], preferred_element_type=_F32)


def _attention(q, qr_pad, k, kr_pad, v):
    return pl.pallas_call(
        _attn_body,
        grid=(B, NH),
        in_specs=[
            pl.BlockSpec((S, Dh), lambda b, h: (b, h)),
            pl.BlockSpec((S, 128), lambda b, h: (b, h)),
            pl.BlockSpec((S, Dh), lambda b, h: (b, h)),
            pl.BlockSpec((S, 128), lambda b, h: (b, 0)),
            pl.BlockSpec((S, Dh), lambda b, h: (b, h)),
        ],
        out_specs=pl.BlockSpec((S, Dh), lambda b, h: (b, h)),
        out_shape=jax.ShapeDtypeStruct((M, QH), _F32),
        compiler_params=pltpu.CompilerParams(vmem_limit_bytes=_VMEM_LIMIT),
    )(q, qr_pad, k, kr_pad, v)


_OP_STEPS = 5


def _outproj_body(s_ref, o_ref, wo_ref, out_ref, snd, rcv, sem_s, sem_r):
    j = pl.program_id(0)
    my_x = lax.axis_index("x")
    my_y = lax.axis_index("y")
    partner = (my_x, 1 - my_y)

    def rdma():
        return pltpu.make_async_remote_copy(
            src_ref=snd, dst_ref=rcv, send_sem=sem_s, recv_sem=sem_r,
            device_id=partner, device_id_type=_MESH)

    @pl.when(j == 0)
    def _():
        out_ref[...] = jnp.dot(o_ref[...], wo_ref[...],
                               preferred_element_type=_F32)

    @pl.when((j > 0) & (j < _OP_STEPS - 1))
    def _():
        out_ref[...] = out_ref[...] + jnp.dot(
            o_ref[...], wo_ref[...], preferred_element_type=_F32)

    @pl.when(j == _OP_STEPS - 1)
    def _():
        snd[...] = out_ref[...].astype(_BF16)
        barrier = pltpu.get_barrier_semaphore()
        pl.semaphore_signal(barrier, inc=1, device_id=partner,
                            device_id_type=_MESH)
        pl.semaphore_wait(barrier, 1)
        r = rdma()
        r.start()
        r.wait()
        out_ref[...] = out_ref[...] + rcv[...].astype(_F32)


def _outproj_allreduce(my_y, o_h, wo):
    nk = _OP_STEPS - 1
    bk = QH // nk
    grid_spec = pltpu.PrefetchScalarGridSpec(
        num_scalar_prefetch=1,
        grid=(_OP_STEPS,),
        in_specs=[
            pl.BlockSpec((M, bk),
                         lambda j, s: (0, jnp.minimum(j, nk - 1))),
            pl.BlockSpec((bk, D),
                         lambda j, s: (s[0] * nk + jnp.minimum(j, nk - 1), 0)),
        ],
        out_specs=pl.BlockSpec((M, D), lambda j, s: (0, 0)),
        scratch_shapes=[
            pltpu.VMEM((M, D), _BF16),
            pltpu.VMEM((M, D), _BF16),
            pltpu.SemaphoreType.DMA,
            pltpu.SemaphoreType.DMA,
        ],
    )
    return pl.pallas_call(
        _outproj_body,
        grid_spec=grid_spec,
        out_shape=jax.ShapeDtypeStruct((M, D), _F32),
        compiler_params=pltpu.CompilerParams(collective_id=1,
                                             vmem_limit_bytes=_VMEM_LIMIT),
    )(_sel(my_y), o_h, wo)


def kernel(x, Wdkv, Wuk, Wuv, Wq, Wqr, Wkr, Wo):
    x2 = x.reshape(M, D)
    my_y = lax.axis_index("y")

    c = _matmul(x2, Wdkv, block_n=Wdkv.shape[1])

    k_h, v_h, q_h = _proj_kv_allreduce(my_y, c, Wuk, Wuv, x2, Wq)
    qr_h = _matmul_half(x2, Wqr, my_y, QRH, block_n=256)
    kr = _matmul(x2, Wkr, block_n=Dr)

    qr_pad = jnp.pad(qr_h.reshape(M, NH, Dr),
                     ((0, 0), (0, 0), (0, 128 - Dr))).reshape(M, NH * 128)
    kr_pad = jnp.pad(kr, ((0, 0), (0, 128 - Dr)))

    o_h = _attention(q_h, qr_pad, k_h, kr_pad, v_h)

    out = _outproj_allreduce(my_y, o_h, Wo)
    return out.reshape(B, S, D)
